# Initial kernel scaffold; baseline (speedup 1.0000x reference)
import jax
import jax.numpy as jnp
from jax.experimental import pallas as pl


def kernel(input_ids, token_type_ids, word_emb, type_emb, W1, b1, W2, b2, W3, b3, pos_gain, gamma, beta):
    raise NotImplementedError("write your pallas kernel here")


# speedup vs baseline: 2.7927x; 2.7927x over previous
import functools

import jax
import jax.numpy as jnp
from jax import lax
from jax.experimental import pallas as pl
from jax.experimental.pallas import tpu as pltpu
from jax.experimental.pallas import tpu_sc as plsc

VOCAB = 30522
H = 1024
MAXPOS = 4096
WIDTH = 256
LEVELS = 16
B = 4
L = 4096
EPS = 1e-12

TOKENS = B * L
NC = 2
NS = 16
NW = NC * NS
PER_W = TOKENS // NW
CHUNK = 32
NCHUNK = PER_W // CHUNK


@functools.cache
def _make_sc_gather():
  @functools.partial(
    pl.kernel,
    mesh=plsc.VectorSubcoreMesh(core_axis_name="c", subcore_axis_name="s"),
    out_type=jax.ShapeDtypeStruct((TOKENS, H), jnp.float32),
    scratch_types=[
        pltpu.VMEM((PER_W,), jnp.int32),
        pltpu.VMEM((CHUNK, H), jnp.float32),
        pltpu.VMEM((CHUNK, H), jnp.float32),
        pltpu.SemaphoreType.DMA,
        pltpu.SemaphoreType.DMA,
        pltpu.SemaphoreType.DMA,
        pltpu.SemaphoreType.DMA,
    ],
  )
  def _sc_gather(idx_hbm, table_hbm, out_hbm, idx_v, rows0, rows1,
                 gsem0, gsem1, osem0, osem1):
    wid = lax.axis_index("s") * NC + lax.axis_index("c")
    base = wid * PER_W
    pltpu.sync_copy(idx_hbm.at[pl.ds(base, PER_W)], idx_v)

    bufs = (rows0, rows1)
    gsems = (gsem0, gsem1)
    osems = (osem0, osem1)

    def gather(c):
        return pltpu.make_async_copy(
            table_hbm.at[idx_v.at[pl.ds(c * CHUNK, CHUNK)]],
            bufs[c % 2],
            gsems[c % 2],
        )

    def writeout(c):
        return pltpu.make_async_copy(
            bufs[c % 2],
            out_hbm.at[pl.ds(base + c * CHUNK, CHUNK)],
            osems[c % 2],
        )

    gather(0).start()
    for c in range(NCHUNK):
        if c + 1 < NCHUNK:
            if c >= 1:
                writeout(c - 1).wait()
            gather(c + 1).start()
        gather(c).wait()
        writeout(c).start()
    writeout(NCHUNK - 2).wait()
    writeout(NCHUNK - 1).wait()

  return _sc_gather


PE_TILE = 512


def _gelu_exact(z):
    return 0.5 * z * (1.0 + lax.erf(z * jnp.float32(0.7071067811865476)))


def _pe_body(w1, b1, w2, b2, w3, b3, gain, out_ref):
    i = pl.program_id(0)
    pos = (i * PE_TILE + lax.broadcasted_iota(jnp.int32, (PE_TILE, 1), 0)
           ).astype(jnp.float32)
    x = pos / jnp.float32(MAXPOS - 1)
    y = x
    cv = jnp.zeros_like(y)
    weight = 0.5
    for _ in range(LEVELS):
        t = jnp.floor(y * 3.0)
        cv = cv + jnp.where(t == 2.0, jnp.float32(weight), 0.0)
        y = y * 3.0 - t
        weight = weight * 0.5
    cv = jnp.clip(cv, 0.0, 1.0)

    h = _gelu_exact(cv * w1[...] + b1[...])
    h = _gelu_exact(
        lax.dot_general(h, w2[...], (((1,), (0,)), ((), ())),
                        precision=lax.Precision.HIGHEST,
                        preferred_element_type=jnp.float32) + b2[...]
    )
    pe = lax.dot_general(h, w3[...], (((1,), (0,)), ((), ())),
                         precision=lax.Precision.HIGHEST,
                         preferred_element_type=jnp.float32) + b3[...]
    out_ref[...] = gain[...] * pe


def _compute_pe(W1, b1, W2, b2, W3, b3, pos_gain):
    return pl.pallas_call(
        _pe_body,
        grid=(L // PE_TILE,),
        in_specs=[
            pl.BlockSpec((1, WIDTH), lambda i: (0, 0)),
            pl.BlockSpec((1, WIDTH), lambda i: (0, 0)),
            pl.BlockSpec((WIDTH, WIDTH), lambda i: (0, 0)),
            pl.BlockSpec((1, WIDTH), lambda i: (0, 0)),
            pl.BlockSpec((WIDTH, H), lambda i: (0, 0)),
            pl.BlockSpec((1, H), lambda i: (0, 0)),
            pl.BlockSpec((1, 1), lambda i: (0, 0)),
        ],
        out_specs=pl.BlockSpec((PE_TILE, H), lambda i: (i, 0)),
        out_shape=jax.ShapeDtypeStruct((L, H), jnp.float32),
    )(W1, b1.reshape(1, WIDTH), W2, b2.reshape(1, WIDTH), W3,
      b3.reshape(1, H), pos_gain.reshape(1, 1))


LN_TILE = 512


def _ln_body(g_ref, pe_ref, tt_ref, te_ref, gamma_ref, beta_ref, out_ref):
    te = te_ref[...]
    t0 = te[0:1, :]
    td = te[1:2, :] - t0
    emb = g_ref[...] + pe_ref[...] + t0 + tt_ref[...] * td
    mean = jnp.mean(emb, axis=1, keepdims=True)
    c = emb - mean
    var = jnp.mean(c * c, axis=1, keepdims=True)
    out_ref[...] = (c / jnp.sqrt(var + EPS)) * gamma_ref[...] + beta_ref[...]


def _fused_ln(gathered, pe, tt_f32, type_emb, gamma, beta):
    nlt = L // LN_TILE
    return pl.pallas_call(
        _ln_body,
        grid=(TOKENS // LN_TILE,),
        in_specs=[
            pl.BlockSpec((LN_TILE, H), lambda t: (t, 0)),
            pl.BlockSpec((LN_TILE, H), lambda t: (t % nlt, 0)),
            pl.BlockSpec((LN_TILE, 1), lambda t: (t, 0)),
            pl.BlockSpec((2, H), lambda t: (0, 0)),
            pl.BlockSpec((1, H), lambda t: (0, 0)),
            pl.BlockSpec((1, H), lambda t: (0, 0)),
        ],
        out_specs=pl.BlockSpec((LN_TILE, H), lambda t: (t, 0)),
        out_shape=jax.ShapeDtypeStruct((TOKENS, H), jnp.float32),
    )(gathered, pe, tt_f32, type_emb, gamma.reshape(1, H), beta.reshape(1, H))


def kernel(input_ids, token_type_ids, word_emb, type_emb, W1, b1, W2, b2,
           W3, b3, pos_gain, gamma, beta):
    idx = input_ids.reshape(TOKENS).astype(jnp.int32)
    gathered = _make_sc_gather()(idx, word_emb)
    pe = _compute_pe(W1, b1, W2, b2, W3, b3, pos_gain)
    tt = token_type_ids.reshape(TOKENS, 1).astype(jnp.float32)
    out = _fused_ln(gathered, pe, tt, type_emb, gamma, beta)
    return out.reshape(B, L, H)
